# Initial kernel scaffold; baseline (speedup 1.0000x reference)
import jax
import jax.numpy as jnp
from jax.experimental import pallas as pl


def kernel(xyz, xyz_nn, points, knn, dists, W1, b1, W2, b2, W3, b3, W4, b4, W5, b5):
    raise NotImplementedError("write your pallas kernel here")


# speedup vs baseline: 10.1203x; 10.1203x over previous
import functools

import jax
import jax.numpy as jnp
from jax import lax
from jax.experimental import pallas as pl
from jax.experimental.pallas import tpu as pltpu
from jax.experimental.pallas import tpu_sc as plsc

_NC = 2
_NS = 16
_CH = 128


def _sc_gather(table, idx2d, e_pad, d_pad, per_w):
  n_chunks = e_pad // _CH
  mesh = plsc.VectorSubcoreMesh(core_axis_name="c", subcore_axis_name="s")

  @functools.partial(
      pl.kernel,
      mesh=mesh,
      compiler_params=pltpu.CompilerParams(use_tc_tiling_on_sc=False),
      out_type=jax.ShapeDtypeStruct((e_pad, d_pad), jnp.float32),
      scratch_types=[
          pltpu.VMEM((_CH,), jnp.int32),
          pltpu.VMEM((_CH, d_pad), jnp.float32),
          pltpu.SemaphoreType.DMA,
      ],
  )
  def gather_kernel(table_hbm, idx_hbm, out_hbm, idx_v, rows_v, sem):
    wid = lax.axis_index("s") * _NC + lax.axis_index("c")

    def body(j, carry):
      row = wid * per_w + j
      pltpu.sync_copy(idx_hbm.at[row], idx_v)
      pltpu.async_copy(table_hbm.at[idx_v], rows_v, sem).wait()
      pltpu.sync_copy(rows_v, out_hbm.at[pl.ds(row * _CH, _CH)])
      return carry

    lax.fori_loop(0, per_w, body, 0)

  return gather_kernel(table, idx2d)


def _tc_compute(g, adj, pts, w1p, w1a, b1, w2p, b2p, w3p, w3a, b3, w4, w5,
                b45, n, k, c, d_pad, blk):
  grid = (n // blk,)
  rows = blk * k

  def body(g_ref, adj_ref, pts_ref, w1p_r, w1a_r, b1_r, w2p_r, b2p_r, w3p_r,
           w3a_r, b3_r, w4_r, w5_r, b45_r, out_ref):
    gv = g_ref[...]
    av = adj_ref[...]
    h = jnp.maximum(
        jnp.dot(gv, w1p_r[...], preferred_element_type=jnp.float32)
        + jnp.dot(av, w1a_r[...], preferred_element_type=jnp.float32)
        + b1_r[...], 0.0)
    dk = jnp.dot(h, w2p_r[...],
                 preferred_element_type=jnp.float32) + b2p_r[...]
    dk3 = dk.reshape(blk, k, d_pad)
    m = jnp.max(dk3, axis=1, keepdims=True)
    e = jnp.exp(dk3 - m)
    s = jnp.sum(e, axis=1, keepdims=True) + 1e-08
    sm = e / s
    g3 = gv.reshape(blk, k, d_pad)
    agg = jnp.sum(sm * g3, axis=1)
    adj3 = av.reshape(blk, k, 4)
    agg_a = jnp.sum(sm[:, :, 0:4] * adj3, axis=1)
    t = jnp.maximum(
        jnp.dot(agg, w3p_r[...], preferred_element_type=jnp.float32)
        + jnp.dot(agg_a, w3a_r[...], preferred_element_type=jnp.float32)
        + b3_r[...], 0.0)
    out_ref[...] = (
        jnp.dot(t, w4_r[...], preferred_element_type=jnp.float32)
        + jnp.dot(pts_ref[...], w5_r[...], preferred_element_type=jnp.float32)
        + b45_r[...])

  full = lambda shape: pl.BlockSpec(shape, lambda i: (0, 0))
  return pl.pallas_call(
      body,
      grid=grid,
      in_specs=[
          pl.BlockSpec((rows, d_pad), lambda i: (i, 0)),
          pl.BlockSpec((rows, 4), lambda i: (i, 0)),
          pl.BlockSpec((blk, c), lambda i: (i, 0)),
          full(w1p.shape), full(w1a.shape), full(b1.shape),
          full(w2p.shape), full(b2p.shape), full(w3p.shape),
          full(w3a.shape), full(b3.shape), full(w4.shape),
          full(w5.shape), full(b45.shape),
      ],
      out_specs=pl.BlockSpec((blk, w4.shape[1]), lambda i: (i, 0)),
      out_shape=jax.ShapeDtypeStruct((n, w4.shape[1]), jnp.float32),
  )(g, adj, pts, w1p, w1a, b1, w2p, b2p, w3p, w3a, b3, w4, w5, b45)


def _pick_block(n):
  for blk in (1000, 800, 625, 500, 400, 250, 200, 125, 100, 50, 40, 25, 20,
              10, 8, 5, 4, 2, 1):
    if n % blk == 0:
      return blk
  return 1


def kernel(xyz, xyz_nn, points, knn, dists, W1, b1, W2, b2, W3, b3, W4, b4,
           W5, b5):
  B, N, K = knn.shape
  C = points.shape[-1]
  din = C + 4
  d_pad = ((din + 15) // 16) * 16
  BN = B * N
  E = BN * K

  f32 = jnp.float32
  table = jnp.concatenate(
      [
          xyz_nn.reshape(BN, 3).astype(f32),
          jnp.zeros((BN, 1), f32),
          points.reshape(BN, C).astype(f32),
          jnp.zeros((BN, d_pad - din), f32),
      ],
      axis=1)

  idx = (knn.astype(jnp.int32)
         + (jnp.arange(B, dtype=jnp.int32) * N)[:, None, None]).reshape(-1)
  per_w = -(-E // (_CH * _NC * _NS))
  e_pad = per_w * _CH * _NC * _NS
  idx = jnp.pad(idx, (0, e_pad - E))
  idx2d = idx.reshape(e_pad // _CH, _CH)

  g = _sc_gather(table, idx2d, e_pad, d_pad, per_w)

  adj = jnp.concatenate(
      [
          jnp.broadcast_to(-xyz[:, :, None, :].astype(f32), (B, N, K, 3)),
          dists[..., None].astype(f32),
      ],
      axis=-1).reshape(E, 4)

  pad_rows = lambda w, r: jnp.pad(w.astype(f32), ((0, r - w.shape[0]), (0, 0)))
  w1p = pad_rows(W1, d_pad)
  w1a = W1[0:4].astype(f32)
  w2p = jnp.pad(W2.astype(f32), ((0, 0), (0, d_pad - din)))
  b2p = jnp.pad(b2.astype(f32), (0, d_pad - din)).reshape(1, d_pad)
  w3p = pad_rows(W3, d_pad)
  w3a = W3[0:4].astype(f32)

  out = _tc_compute(
      g, adj, points.reshape(BN, C).astype(f32),
      w1p, w1a, b1.astype(f32).reshape(1, -1),
      w2p, b2p, w3p, w3a, b3.astype(f32).reshape(1, -1),
      W4.astype(f32), W5.astype(f32),
      (b4 + b5).astype(f32).reshape(1, -1),
      BN, K, C, d_pad, _pick_block(BN))

  return (xyz, out.reshape(B, N, -1))


# speedup vs baseline: 12.4549x; 1.2307x over previous
import functools

import jax
import jax.numpy as jnp
from jax import lax
from jax.experimental import pallas as pl
from jax.experimental.pallas import tpu as pltpu
from jax.experimental.pallas import tpu_sc as plsc

_NC = 2
_NS = 16
_CH = 128
_DP = 128


def _sc_gather(table, idx2d, dists_flat, e_pad, per_w):
  mesh = plsc.VectorSubcoreMesh(core_axis_name="c", subcore_axis_name="s")

  @functools.partial(
      pl.kernel,
      mesh=mesh,
      compiler_params=pltpu.CompilerParams(needs_layout_passes=False),
      out_type=jax.ShapeDtypeStruct((e_pad, _DP), jnp.float32),
      scratch_types=[
          pltpu.VMEM((_CH,), jnp.int32),
          pltpu.VMEM((_CH,), jnp.float32),
          pltpu.VMEM((_CH, _DP), jnp.float32),
          pltpu.SemaphoreType.DMA,
      ],
  )
  def gather_kernel(table_hbm, idx_hbm, dist_hbm, out_hbm, idx_v, dist_v,
                    rows_v, sem):
    wid = lax.axis_index("s") * _NC + lax.axis_index("c")
    lanes = lax.iota(jnp.int32, 16)
    col3 = jnp.full((16,), 3, jnp.int32)

    def body(j, carry):
      row = wid * per_w + j
      pltpu.sync_copy(idx_hbm.at[row], idx_v)
      pltpu.sync_copy(dist_hbm.at[pl.ds(row * _CH, _CH)], dist_v)
      pltpu.async_copy(table_hbm.at[idx_v], rows_v, sem).wait()
      for grp in range(_CH // 16):
        dvec = dist_v[pl.ds(grp * 16, 16)]
        plsc.store_scatter(rows_v, [grp * 16 + lanes, col3], dvec)
      pltpu.sync_copy(rows_v, out_hbm.at[pl.ds(row * _CH, _CH)])
      return carry

    lax.fori_loop(0, per_w, body, 0)

  return gather_kernel(table, idx2d, dists_flat)


def _tc_compute(g, xyzn, pts, w1p, w1a, b1, w2p, b2p, w3p, w3a, b3, w4, w5,
                b45, n, k, c, blk):
  grid = (n // blk,)
  rows = blk * k

  def body(g_ref, xyzn_ref, pts_ref, w1p_r, w1a_r, b1_r, w2p_r, b2p_r, w3p_r,
           w3a_r, b3_r, w4_r, w5_r, b45_r, out_ref):
    gv = g_ref[...]
    x4 = xyzn_ref[...]
    u = jnp.dot(x4, w1a_r[...],
                preferred_element_type=jnp.float32) + b1_r[...]
    gw = jnp.dot(gv, w1p_r[...],
                 preferred_element_type=jnp.float32)
    h3 = jnp.maximum(gw.reshape(blk, k, 32) + u[:, None, :], 0.0)
    h = h3.reshape(rows, 32)
    dk = jnp.dot(h, w2p_r[...],
                 preferred_element_type=jnp.float32) + b2p_r[...]
    dk3 = dk.reshape(blk, k, _DP)
    m = jnp.max(dk3, axis=1, keepdims=True)
    e = jnp.exp(dk3 - m)
    s = jnp.sum(e, axis=1, keepdims=True) + 1e-08
    sm = e / s
    g3 = gv.reshape(blk, k, _DP)
    agg = jnp.sum(sm * g3, axis=1)
    s4 = jnp.sum(sm[:, :, 0:4], axis=1)
    t = jnp.maximum(
        jnp.dot(agg, w3p_r[...], preferred_element_type=jnp.float32)
        + jnp.dot(s4 * x4, w3a_r[...], preferred_element_type=jnp.float32)
        + b3_r[...], 0.0)
    out_ref[...] = (
        jnp.dot(t, w4_r[...], preferred_element_type=jnp.float32)
        + jnp.dot(pts_ref[...], w5_r[...], preferred_element_type=jnp.float32)
        + b45_r[...])

  full = lambda shape: pl.BlockSpec(shape, lambda i: (0, 0))
  return pl.pallas_call(
      body,
      grid=grid,
      in_specs=[
          pl.BlockSpec((rows, _DP), lambda i: (i, 0)),
          pl.BlockSpec((blk, 4), lambda i: (i, 0)),
          pl.BlockSpec((blk, c), lambda i: (i, 0)),
          full(w1p.shape), full(w1a.shape), full(b1.shape),
          full(w2p.shape), full(b2p.shape), full(w3p.shape),
          full(w3a.shape), full(b3.shape), full(w4.shape),
          full(w5.shape), full(b45.shape),
      ],
      out_specs=pl.BlockSpec((blk, w4.shape[1]), lambda i: (i, 0)),
      out_shape=jax.ShapeDtypeStruct((n, w4.shape[1]), jnp.float32),
  )(g, xyzn, pts, w1p, w1a, b1, w2p, b2p, w3p, w3a, b3, w4, w5, b45)


def _pick_block(n):
  for blk in (1000, 800, 625, 500, 400, 250, 200, 125, 100, 50, 40, 25, 20,
              10, 8, 5, 4, 2, 1):
    if n % blk == 0:
      return blk
  return 1


def kernel(xyz, xyz_nn, points, knn, dists, W1, b1, W2, b2, W3, b3, W4, b4,
           W5, b5):
  B, N, K = knn.shape
  C = points.shape[-1]
  din = C + 4
  BN = B * N
  E = BN * K

  f32 = jnp.float32
  table = jnp.concatenate(
      [
          xyz_nn.reshape(BN, 3).astype(f32),
          jnp.zeros((BN, 1), f32),
          points.reshape(BN, C).astype(f32),
          jnp.zeros((BN, _DP - din), f32),
      ],
      axis=1)

  idx = (knn.astype(jnp.int32)
         + (jnp.arange(B, dtype=jnp.int32) * N)[:, None, None]).reshape(-1)
  per_w = -(-E // (_CH * _NC * _NS))
  e_pad = per_w * _CH * _NC * _NS
  idx = jnp.pad(idx, (0, e_pad - E))
  idx2d = idx.reshape(e_pad // _CH, _CH)
  dflat = jnp.pad(dists.astype(f32).reshape(-1), (0, e_pad - E))

  g = _sc_gather(table, idx2d, dflat, e_pad, per_w)

  xyzn = jnp.concatenate(
      [-xyz.reshape(BN, 3).astype(f32), jnp.zeros((BN, 1), f32)], axis=1)

  pad_rows = lambda w, r: jnp.pad(w.astype(f32), ((0, r - w.shape[0]), (0, 0)))
  w1p = pad_rows(W1, _DP)
  w1a = W1[0:4].astype(f32)
  w2p = jnp.pad(W2.astype(f32), ((0, 0), (0, _DP - din)))
  b2p = jnp.pad(b2.astype(f32), (0, _DP - din)).reshape(1, _DP)
  w3p = pad_rows(W3, _DP)
  w3a = W3[0:4].astype(f32)

  out = _tc_compute(
      g, xyzn, points.reshape(BN, C).astype(f32),
      w1p, w1a, b1.astype(f32).reshape(1, -1),
      w2p, b2p, w3p, w3a, b3.astype(f32).reshape(1, -1),
      W4.astype(f32), W5.astype(f32),
      (b4 + b5).astype(f32).reshape(1, -1),
      BN, K, C, _pick_block(BN))

  return (xyz, out.reshape(B, N, -1))


# speedup vs baseline: 15.5225x; 1.2463x over previous
import functools

import jax
import jax.numpy as jnp
from jax import lax
from jax.experimental import pallas as pl
from jax.experimental.pallas import tpu as pltpu
from jax.experimental.pallas import tpu_sc as plsc

_NC = 2
_NS = 16
_CH = 128
_DP = 128


def _sc_gather(table, idx2d, dists_flat, e_pad, per_w):
  assert per_w % 2 == 0
  mesh = plsc.VectorSubcoreMesh(core_axis_name="c", subcore_axis_name="s")

  @functools.partial(
      pl.kernel,
      mesh=mesh,
      compiler_params=pltpu.CompilerParams(needs_layout_passes=False),
      out_type=jax.ShapeDtypeStruct((e_pad, _DP), jnp.float32),
      scratch_types=[
          pltpu.VMEM((2, _CH), jnp.int32),
          pltpu.VMEM((2, _CH), jnp.float32),
          pltpu.VMEM((2, _CH, _DP), jnp.float32),
          pltpu.SemaphoreType.DMA, pltpu.SemaphoreType.DMA,
          pltpu.SemaphoreType.DMA, pltpu.SemaphoreType.DMA,
          pltpu.SemaphoreType.DMA, pltpu.SemaphoreType.DMA,
          pltpu.SemaphoreType.DMA,
      ],
  )
  def gather_kernel(table_hbm, idx_hbm, dist_hbm, out_hbm, idx_v, dist_v,
                    rows_v, si0, si1, sd0, sd1, so0, so1, sg):
    wid = lax.axis_index("s") * _NC + lax.axis_index("c")
    base = wid * per_w
    lanes = lax.iota(jnp.int32, 16)
    col3 = jnp.full((16,), 3, jnp.int32)
    si = (si0, si1)
    sd = (sd0, sd1)
    so = (so0, so1)

    def in_copies(row, b):
      return (
          pltpu.make_async_copy(idx_hbm.at[row], idx_v.at[b], si[b]),
          pltpu.make_async_copy(dist_hbm.at[pl.ds(row * _CH, _CH)],
                                dist_v.at[b], sd[b]),
      )

    def out_copy(row, b):
      return pltpu.make_async_copy(rows_v.at[b],
                                   out_hbm.at[pl.ds(row * _CH, _CH)], so[b])

    for cp in in_copies(base, 0):
      cp.start()

    def body(jj, carry):
      for b in (0, 1):
        j = jj * 2 + b
        row = base + j

        @pl.when(jj >= 1)
        def _wait_old_out():
          out_copy(row - 2, b).wait()

        for cp in in_copies(row, b):
          cp.wait()

        @pl.when(j + 1 < per_w)
        def _start_next_in():
          for cp in in_copies(row + 1, 1 - b):
            cp.start()

        pltpu.async_copy(table_hbm.at[idx_v.at[b]], rows_v.at[b], sg).wait()
        for grp in range(_CH // 16):
          dvec = dist_v[b, pl.ds(grp * 16, 16)]
          plsc.store_scatter(rows_v.at[b], [grp * 16 + lanes, col3], dvec)
        out_copy(row, b).start()
      return carry

    lax.fori_loop(0, per_w // 2, body, 0)
    out_copy(base + per_w - 2, 0).wait()
    out_copy(base + per_w - 1, 1).wait()

  return gather_kernel(table, idx2d, dists_flat)


def _tc_compute(g, xyzn, pts, w1p, w1a, b1, w2p, b2p, w3p, w3a, b3, w4, w5,
                b45, n, k, c, blk):
  grid = (n // blk,)
  rows = blk * k

  def body(g_ref, xyzn_ref, pts_ref, w1p_r, w1a_r, b1_r, w2p_r, b2p_r, w3p_r,
           w3a_r, b3_r, w4_r, w5_r, b45_r, out_ref):
    gv = g_ref[...]
    x4 = xyzn_ref[...]
    u = jnp.dot(x4, w1a_r[...],
                preferred_element_type=jnp.float32) + b1_r[...]
    gw = jnp.dot(gv, w1p_r[...],
                 preferred_element_type=jnp.float32)
    h3 = jnp.maximum(gw.reshape(blk, k, 32) + u[:, None, :], 0.0)
    h = h3.reshape(rows, 32)
    dk = jnp.dot(h, w2p_r[...],
                 preferred_element_type=jnp.float32) + b2p_r[...]
    dk3 = dk.reshape(blk, k, _DP)
    m = jnp.max(dk3, axis=1, keepdims=True)
    e = jnp.exp(dk3 - m)
    s = jnp.sum(e, axis=1, keepdims=True) + 1e-08
    sm = e / s
    g3 = gv.reshape(blk, k, _DP)
    agg = jnp.sum(sm * g3, axis=1)
    s4 = jnp.sum(sm[:, :, 0:4], axis=1)
    t = jnp.maximum(
        jnp.dot(agg, w3p_r[...], preferred_element_type=jnp.float32)
        + jnp.dot(s4 * x4, w3a_r[...], preferred_element_type=jnp.float32)
        + b3_r[...], 0.0)
    out_ref[...] = (
        jnp.dot(t, w4_r[...], preferred_element_type=jnp.float32)
        + jnp.dot(pts_ref[...], w5_r[...], preferred_element_type=jnp.float32)
        + b45_r[...])

  full = lambda shape: pl.BlockSpec(shape, lambda i: (0, 0))
  return pl.pallas_call(
      body,
      grid=grid,
      in_specs=[
          pl.BlockSpec((rows, _DP), lambda i: (i, 0)),
          pl.BlockSpec((blk, 4), lambda i: (i, 0)),
          pl.BlockSpec((blk, c), lambda i: (i, 0)),
          full(w1p.shape), full(w1a.shape), full(b1.shape),
          full(w2p.shape), full(b2p.shape), full(w3p.shape),
          full(w3a.shape), full(b3.shape), full(w4.shape),
          full(w5.shape), full(b45.shape),
      ],
      out_specs=pl.BlockSpec((blk, w4.shape[1]), lambda i: (i, 0)),
      out_shape=jax.ShapeDtypeStruct((n, w4.shape[1]), jnp.float32),
  )(g, xyzn, pts, w1p, w1a, b1, w2p, b2p, w3p, w3a, b3, w4, w5, b45)


def _pick_block(n):
  for blk in (1000, 800, 625, 500, 400, 250, 200, 125, 100, 50, 40, 25, 20,
              10, 8, 5, 4, 2, 1):
    if n % blk == 0:
      return blk
  return 1


def kernel(xyz, xyz_nn, points, knn, dists, W1, b1, W2, b2, W3, b3, W4, b4,
           W5, b5):
  B, N, K = knn.shape
  C = points.shape[-1]
  din = C + 4
  BN = B * N
  E = BN * K

  f32 = jnp.float32
  table = jnp.concatenate(
      [
          xyz_nn.reshape(BN, 3).astype(f32),
          jnp.zeros((BN, 1), f32),
          points.reshape(BN, C).astype(f32),
          jnp.zeros((BN, _DP - din), f32),
      ],
      axis=1)

  idx = (knn.astype(jnp.int32)
         + (jnp.arange(B, dtype=jnp.int32) * N)[:, None, None]).reshape(-1)
  per_w = -(-E // (_CH * _NC * _NS))
  per_w = per_w + (per_w % 2)
  e_pad = per_w * _CH * _NC * _NS
  idx = jnp.pad(idx, (0, e_pad - E))
  idx2d = idx.reshape(e_pad // _CH, _CH)
  dflat = jnp.pad(dists.astype(f32).reshape(-1), (0, e_pad - E))

  g = _sc_gather(table, idx2d, dflat, e_pad, per_w)

  xyzn = jnp.concatenate(
      [-xyz.reshape(BN, 3).astype(f32), jnp.zeros((BN, 1), f32)], axis=1)

  pad_rows = lambda w, r: jnp.pad(w.astype(f32), ((0, r - w.shape[0]), (0, 0)))
  w1p = pad_rows(W1, _DP)
  w1a = W1[0:4].astype(f32)
  w2p = jnp.pad(W2.astype(f32), ((0, 0), (0, _DP - din)))
  b2p = jnp.pad(b2.astype(f32), (0, _DP - din)).reshape(1, _DP)
  w3p = pad_rows(W3, _DP)
  w3a = W3[0:4].astype(f32)

  out = _tc_compute(
      g, xyzn, points.reshape(BN, C).astype(f32),
      w1p, w1a, b1.astype(f32).reshape(1, -1),
      w2p, b2p, w3p, w3a, b3.astype(f32).reshape(1, -1),
      W4.astype(f32), W5.astype(f32),
      (b4 + b5).astype(f32).reshape(1, -1),
      BN, K, C, _pick_block(BN))

  return (xyz, out.reshape(B, N, -1))


# speedup vs baseline: 17.2804x; 1.1133x over previous
import functools

import jax
import jax.numpy as jnp
from jax import lax
from jax.experimental import pallas as pl
from jax.experimental.pallas import tpu as pltpu
from jax.experimental.pallas import tpu_sc as plsc

_NC = 2
_NS = 16
_CH = 128
_DP = 128


def _sc_gather(table, idx2d, dists_flat, e_pad, per_w):
  assert per_w % 2 == 0
  mesh = plsc.VectorSubcoreMesh(core_axis_name="c", subcore_axis_name="s")

  @functools.partial(
      pl.kernel,
      mesh=mesh,
      compiler_params=pltpu.CompilerParams(needs_layout_passes=False),
      out_type=jax.ShapeDtypeStruct((e_pad, _DP), jnp.float32),
      scratch_types=[
          pltpu.VMEM((2, _CH), jnp.int32),
          pltpu.VMEM((2, _CH), jnp.float32),
          pltpu.VMEM((2, _CH, _DP), jnp.float32),
          pltpu.SemaphoreType.DMA, pltpu.SemaphoreType.DMA,
          pltpu.SemaphoreType.DMA, pltpu.SemaphoreType.DMA,
          pltpu.SemaphoreType.DMA, pltpu.SemaphoreType.DMA,
          pltpu.SemaphoreType.DMA,
      ],
  )
  def gather_kernel(table_hbm, idx_hbm, dist_hbm, out_hbm, idx_v, dist_v,
                    rows_v, si0, si1, sd0, sd1, so0, so1, sg):
    wid = lax.axis_index("s") * _NC + lax.axis_index("c")
    base = wid * per_w
    lanes = lax.iota(jnp.int32, 16)
    col3 = jnp.full((16,), 3, jnp.int32)
    si = (si0, si1)
    sd = (sd0, sd1)
    so = (so0, so1)

    def in_copies(row, b):
      return (
          pltpu.make_async_copy(idx_hbm.at[row], idx_v.at[b], si[b]),
          pltpu.make_async_copy(dist_hbm.at[pl.ds(row * _CH, _CH)],
                                dist_v.at[b], sd[b]),
      )

    def out_copy(row, b):
      return pltpu.make_async_copy(rows_v.at[b],
                                   out_hbm.at[pl.ds(row * _CH, _CH)], so[b])

    for cp in in_copies(base, 0):
      cp.start()

    def body(jj, carry):
      for b in (0, 1):
        j = jj * 2 + b
        row = base + j

        @pl.when(jj >= 1)
        def _wait_old_out():
          out_copy(row - 2, b).wait()

        for cp in in_copies(row, b):
          cp.wait()

        @pl.when(j + 1 < per_w)
        def _start_next_in():
          for cp in in_copies(row + 1, 1 - b):
            cp.start()

        pltpu.async_copy(table_hbm.at[idx_v.at[b]], rows_v.at[b], sg).wait()
        for grp in range(_CH // 16):
          dvec = dist_v[b, pl.ds(grp * 16, 16)]
          plsc.store_scatter(rows_v.at[b], [grp * 16 + lanes, col3], dvec)
        out_copy(row, b).start()
      return carry

    lax.fori_loop(0, per_w // 2, body, 0)
    out_copy(base + per_w - 2, 0).wait()
    out_copy(base + per_w - 1, 1).wait()

  return gather_kernel(table, idx2d, dists_flat)


def _tc_compute(g, xyzn, pts, w1p, w1a, b1, w2p, b2p, w3p, w3a, b3, w4, w5,
                b45, n, k, c, blk):
  grid = (n // blk,)
  rows = blk * k

  def body(g_ref, xyzn_ref, pts_ref, w1p_r, w1a_r, b1_r, w2p_r, b2p_r, w3p_r,
           w3a_r, b3_r, w4_r, w5_r, b45_r, out_ref):
    gv = g_ref[...]
    x4 = xyzn_ref[...]
    u = jnp.dot(x4, w1a_r[...],
                preferred_element_type=jnp.float32) + b1_r[...]
    gw = jnp.dot(gv, w1p_r[...],
                 preferred_element_type=jnp.float32)
    h3 = jnp.maximum(gw.reshape(blk, k, 32) + u[:, None, :], 0.0)
    h = h3.reshape(rows, 32)
    dk = jnp.dot(h, w2p_r[...],
                 preferred_element_type=jnp.float32) + b2p_r[...]
    dk3 = dk.reshape(blk, k, _DP)
    m = jnp.max(dk3, axis=1, keepdims=True)
    e = jnp.exp(dk3 - m)
    s = jnp.sum(e, axis=1, keepdims=True) + 1e-08
    sm = e / s
    g3 = gv.reshape(blk, k, _DP)
    agg = jnp.sum(sm * g3, axis=1)
    s4 = jnp.sum(sm[:, :, 0:4], axis=1)
    t = jnp.maximum(
        jnp.dot(agg, w3p_r[...], preferred_element_type=jnp.float32)
        + jnp.dot(s4 * x4, w3a_r[...], preferred_element_type=jnp.float32)
        + b3_r[...], 0.0)
    out_ref[...] = (
        jnp.dot(t, w4_r[...], preferred_element_type=jnp.float32)
        + jnp.dot(pts_ref[...], w5_r[...], preferred_element_type=jnp.float32)
        + b45_r[...])

  full = lambda shape: pl.BlockSpec(shape, lambda i: (0, 0))
  return pl.pallas_call(
      body,
      grid=grid,
      in_specs=[
          pl.BlockSpec((rows, _DP), lambda i: (i, 0)),
          pl.BlockSpec((blk, 4), lambda i: (i, 0)),
          pl.BlockSpec((blk, c), lambda i: (i, 0)),
          full(w1p.shape), full(w1a.shape), full(b1.shape),
          full(w2p.shape), full(b2p.shape), full(w3p.shape),
          full(w3a.shape), full(b3.shape), full(w4.shape),
          full(w5.shape), full(b45.shape),
      ],
      out_specs=pl.BlockSpec((blk, w4.shape[1]), lambda i: (i, 0)),
      out_shape=jax.ShapeDtypeStruct((n, w4.shape[1]), jnp.float32),
  )(g, xyzn, pts, w1p, w1a, b1, w2p, b2p, w3p, w3a, b3, w4, w5, b45)


def _pick_block(n):
  for blk in (1000, 800, 625, 500, 400, 250, 200, 125, 100, 50, 40, 25, 20,
              10, 8, 5, 4, 2, 1):
    if n % blk == 0:
      return blk
  return 1


def kernel(xyz, xyz_nn, points, knn, dists, W1, b1, W2, b2, W3, b3, W4, b4,
           W5, b5):
  B, N, K = knn.shape
  C = points.shape[-1]
  din = C + 4
  BN = B * N
  E = BN * K

  f32 = jnp.float32
  table = jnp.concatenate(
      [
          xyz_nn.reshape(BN, 3).astype(f32),
          jnp.zeros((BN, 1), f32),
          points.reshape(BN, C).astype(f32),
          jnp.zeros((BN, _DP - din), f32),
      ],
      axis=1)

  idx = (knn.astype(jnp.int32)
         + (jnp.arange(B, dtype=jnp.int32) * N)[:, None, None]).reshape(-1)

  n_sl = 4
  while BN % n_sl or (BN // n_sl) % 8:
    n_sl //= 2
  bn_s = BN // n_sl
  e_s = bn_s * K
  per_w = -(-e_s // (_CH * _NC * _NS))
  per_w = per_w + (per_w % 2)
  e_pad = per_w * _CH * _NC * _NS
  idx_sl = jnp.pad(idx.reshape(n_sl, e_s), ((0, 0), (0, e_pad - e_s)))
  d_sl = jnp.pad(dists.astype(f32).reshape(n_sl, e_s),
                 ((0, 0), (0, e_pad - e_s)))

  xyzn = jnp.concatenate(
      [-xyz.reshape(BN, 3).astype(f32), jnp.zeros((BN, 1), f32)], axis=1)

  pad_rows = lambda w, r: jnp.pad(w.astype(f32), ((0, r - w.shape[0]), (0, 0)))
  w1p = pad_rows(W1, _DP)
  w1a = W1[0:4].astype(f32)
  w2p = jnp.pad(W2.astype(f32), ((0, 0), (0, _DP - din)))
  b2p = jnp.pad(b2.astype(f32), (0, _DP - din)).reshape(1, _DP)
  w3p = pad_rows(W3, _DP)
  w3a = W3[0:4].astype(f32)

  pts = points.reshape(BN, C).astype(f32)
  b1r = b1.astype(f32).reshape(1, -1)
  b3r = b3.astype(f32).reshape(1, -1)
  b45 = (b4 + b5).astype(f32).reshape(1, -1)
  w4 = W4.astype(f32)
  w5 = W5.astype(f32)
  blk = _pick_block(bn_s)

  outs = []
  for i in range(n_sl):
    g = _sc_gather(table, idx_sl[i].reshape(e_pad // _CH, _CH), d_sl[i],
                   e_pad, per_w)
    sl = slice(i * bn_s, (i + 1) * bn_s)
    outs.append(_tc_compute(
        g, xyzn[sl], pts[sl], w1p, w1a, b1r, w2p, b2p, w3p, w3a, b3r,
        w4, w5, b45, bn_s, K, C, blk))
  out = jnp.concatenate(outs, axis=0)

  return (xyz, out.reshape(B, N, -1))
